# baseline (speedup 1.0000x reference)
import functools

import jax
import jax.numpy as jnp
from jax import lax
from jax.experimental import pallas as pl
from jax.experimental.pallas import tpu as pltpu
from jax.experimental.pallas import tpu_sc as plsc

N = 10000
E = 320000
D = 128
H = 128
OUT = 64
PDIM = 32
TEMP = 0.5

NP = 10240
NTILES = 16
TPW = NP // NTILES
CH = 128
NCH = 160
EPT = NCH * CH
EP = EPT * NTILES

def _sc_mesh():
  return plsc.VectorSubcoreMesh(
      core_axis_name="c", subcore_axis_name="s", num_cores=2, num_subcores=16)


_DEG_LAG = 4


@functools.cache
def _make_deg_kernel():
  return functools.partial(
      pl.kernel,
      out_type=(jax.ShapeDtypeStruct((NP, 16), jnp.float32),) * 4,
      mesh=_sc_mesh(),
      scratch_types=(
          pltpu.VMEM_SHARED((NP, 16), jnp.float32),
          pltpu.VMEM_SHARED((NP, 16), jnp.float32),
          pltpu.VMEM((NCH, 2, CH), jnp.int32),
          pltpu.VMEM((CH, 16), jnp.float32),
          pltpu.SemaphoreType.DMA,
      ),
  )(_deg_body)


def _deg_body(e1, e2, zsmall, ones_h,
              ds1_o, dd1_o, ds2_o, dd2_o,
              acc_s, acc_d, idxv, onesv, sem):
  c = lax.axis_index("c")
  s = lax.axis_index("s")
  pltpu.sync_copy(zsmall.at[pl.ds(s * TPW, TPW)], acc_s.at[pl.ds(s * TPW, TPW)])
  pltpu.sync_copy(zsmall.at[pl.ds(s * TPW, TPW)], acc_d.at[pl.ds(s * TPW, TPW)])
  pltpu.sync_copy(ones_h, onesv)

  def body(e_h):
    pltpu.sync_copy(e_h.at[s], idxv)
    plsc.subcore_barrier()

    def drain(j):
      pltpu.make_async_copy(onesv, acc_s.at[idxv.at[j, 0]], sem).wait()
      pltpu.make_async_copy(onesv, acc_d.at[idxv.at[j, 1]], sem).wait()

    def step(j, carry):
      pltpu.async_copy(onesv, acc_s.at[idxv.at[j, 0]], sem, add=True)
      pltpu.async_copy(onesv, acc_d.at[idxv.at[j, 1]], sem, add=True)
      pl.when(j >= _DEG_LAG)(lambda: drain(j - _DEG_LAG))
      return carry

    lax.fori_loop(0, NCH, step, 0)

    def tail(j, carry):
      drain(j)
      return carry

    lax.fori_loop(NCH - _DEG_LAG, NCH, tail, 0)

  pl.when(c == 0)(lambda: body(e1))
  pl.when(c == 1)(lambda: body(e2))
  plsc.subcore_barrier()

  def copyout(o_s, o_d):
    pltpu.sync_copy(acc_s.at[pl.ds(s * TPW, TPW)], o_s.at[pl.ds(s * TPW, TPW)])
    pltpu.sync_copy(acc_d.at[pl.ds(s * TPW, TPW)], o_d.at[pl.ds(s * TPW, TPW)])

  pl.when(c == 0)(lambda: copyout(ds1_o, dd1_o))
  pl.when(c == 1)(lambda: copyout(ds2_o, dd2_o))


_NIDX = 8
_NROW = 4
FH = 64


@functools.cache
def _make_agg_kernel():
  return functools.partial(
      pl.kernel,
      out_type=(jax.ShapeDtypeStruct((NP, FH), jnp.float32),) * 2,
      mesh=_sc_mesh(),
      scratch_types=(
          (pltpu.VMEM_SHARED((NP, FH), jnp.float32),
           pltpu.VMEM_SHARED((NP, FH), jnp.float32))
          + tuple(pltpu.VMEM((2, CH), jnp.int32) for _ in range(_NIDX))
          + tuple(pltpu.VMEM((CH, FH), jnp.float32) for _ in range(_NROW))
          + tuple(pltpu.SemaphoreType.DMA for _ in range(_NIDX + _NROW))
      ),
      compiler_params=pltpu.CompilerParams(use_tc_tiling_on_sc=False),
  )(_agg_body)


def _agg_body(hs1, e1, hs2, e2, zbig, out1, out2, tab, acc, *scratch):
  idxb = scratch[:_NIDX]
  bufs = scratch[_NIDX:_NIDX + _NROW]
  sem_i = scratch[_NIDX + _NROW:2 * _NIDX + _NROW]
  sem_g = scratch[2 * _NIDX + _NROW:]
  c = lax.axis_index("c")
  s = lax.axis_index("s")
  rows = pl.ds(s * TPW, TPW)
  pltpu.sync_copy(zbig, acc.at[rows])

  def body(hs, e_h):
    pltpu.sync_copy(hs.at[rows], tab.at[rows])
    plsc.subcore_barrier()
    for b in range(_NIDX):
      pltpu.async_copy(e_h.at[s, b], idxb[b], sem_i[b])
    for j in range(_NROW):
      pltpu.make_async_copy(e_h.at[s, j], idxb[j], sem_i[j]).wait()
      pltpu.async_copy(tab.at[idxb[j].at[0]], bufs[j], sem_g[j])

    def step(q, carry):
      for b in range(_NIDX):
        j = q * _NIDX + b
        rb = b % _NROW
        pltpu.make_async_copy(tab.at[idxb[b].at[0]], bufs[rb], sem_g[rb]).wait()
        pltpu.sync_copy(bufs[rb], acc.at[idxb[b].at[1]], add=True)

        def fire_idx(b=b, j=j):
          pltpu.async_copy(e_h.at[s, j + _NIDX], idxb[b], sem_i[b])

        pl.when(j + _NIDX < NCH)(fire_idx)

        def fire_gather(b=b, j=j, rb=rb):
          b2 = (b + _NROW) % _NIDX
          pltpu.make_async_copy(e_h.at[s, j + _NROW], idxb[b2],
                                sem_i[b2]).wait()
          pltpu.async_copy(tab.at[idxb[b2].at[0]], bufs[rb], sem_g[rb])

        pl.when(j + _NROW < NCH)(fire_gather)
      return carry

    lax.fori_loop(0, NCH // _NIDX, step, 0)

  pl.when(c == 0)(lambda: body(hs1, e1))
  pl.when(c == 1)(lambda: body(hs2, e2))
  plsc.subcore_barrier()

  pl.when(c == 0)(lambda: pltpu.sync_copy(acc.at[rows], out1.at[rows]))
  pl.when(c == 1)(lambda: pltpu.sync_copy(acc.at[rows], out2.at[rows]))


def _rsqdeg(deg_blk):
  return lax.rsqrt(jnp.maximum(deg_blk[:, 0:1], 1.0))


def _scale_body(x_ref, deg_ref, oa_ref, ob_ref):
  h = x_ref[...] * _rsqdeg(deg_ref[...])
  oa_ref[...] = h[:, :FH]
  ob_ref[...] = h[:, FH:]


def _scale(x, deg):
  R = 2048
  return pl.pallas_call(
      _scale_body,
      grid=(NP // R,),
      in_specs=[pl.BlockSpec((R, D), lambda i: (i, 0)),
                pl.BlockSpec((R, 16), lambda i: (i, 0))],
      out_specs=[pl.BlockSpec((R, FH), lambda i: (i, 0)),
                 pl.BlockSpec((R, FH), lambda i: (i, 0))],
      out_shape=[jax.ShapeDtypeStruct((NP, FH), jnp.float32),
                 jax.ShapeDtypeStruct((NP, FH), jnp.float32)],
  )(x, deg)


def _layer1_body(aa_ref, ab_ref, dd_ref, ds_ref, w_ref, b_ref, w2_ref, o_ref):
  nd = _rsqdeg(dd_ref[...])
  h = (jnp.dot(aa_ref[...] * nd, w_ref[:FH, :],
               preferred_element_type=jnp.float32)
       + jnp.dot(ab_ref[...] * nd, w_ref[FH:, :],
                 preferred_element_type=jnp.float32)
       + b_ref[...])
  h = jnp.maximum(h, 0.0) * _rsqdeg(ds_ref[...])
  o_ref[...] = jnp.dot(h, w2_ref[...], preferred_element_type=jnp.float32)


def _layer1(agga, aggb, dd, ds, w, b, w2):
  R = 1024
  return pl.pallas_call(
      _layer1_body,
      grid=(NP // R,),
      in_specs=[pl.BlockSpec((R, FH), lambda i: (i, 0)),
                pl.BlockSpec((R, FH), lambda i: (i, 0)),
                pl.BlockSpec((R, 16), lambda i: (i, 0)),
                pl.BlockSpec((R, 16), lambda i: (i, 0)),
                pl.BlockSpec((D, H), lambda i: (0, 0)),
                pl.BlockSpec((1, H), lambda i: (0, 0)),
                pl.BlockSpec((H, OUT), lambda i: (0, 0))],
      out_specs=pl.BlockSpec((R, OUT), lambda i: (i, 0)),
      out_shape=jax.ShapeDtypeStruct((NP, OUT), jnp.float32),
  )(agga, aggb, dd, ds, w, b, w2)


_FR = 1024


def _final_body(agg_ref, dd_ref, b2_ref, p1w_ref, p1b_ref,
                p2w_ref, p2b_ref, o_ref):
  h = agg_ref[...] * _rsqdeg(dd_ref[...]) + b2_ref[...]
  h = jnp.maximum(h, 0.0)
  u = jnp.dot(h, p1w_ref[...], preferred_element_type=jnp.float32) + p1b_ref[...]
  u = jnp.where(u > 0.0, u, jnp.exp(u) - 1.0)
  z = jnp.dot(u, p2w_ref[...], preferred_element_type=jnp.float32) + p2b_ref[...]
  nrm = jnp.maximum(jnp.sqrt(jnp.sum(z * z, axis=1, keepdims=True)), 1e-12)
  row = lax.broadcasted_iota(jnp.int32, (_FR, OUT), 0) + pl.program_id(0) * _FR
  o_ref[...] = jnp.where(row < N, z / nrm, 0.0)


def _final(agg, dd, b2, p1w, p1b, p2w, p2b):
  R = _FR
  return pl.pallas_call(
      _final_body,
      grid=(NP // R,),
      in_specs=[pl.BlockSpec((R, OUT), lambda i: (i, 0)),
                pl.BlockSpec((R, 16), lambda i: (i, 0)),
                pl.BlockSpec((1, OUT), lambda i: (0, 0)),
                pl.BlockSpec((OUT, PDIM), lambda i: (0, 0)),
                pl.BlockSpec((1, PDIM), lambda i: (0, 0)),
                pl.BlockSpec((PDIM, OUT), lambda i: (0, 0)),
                pl.BlockSpec((1, OUT), lambda i: (0, 0))],
      out_specs=pl.BlockSpec((R, OUT), lambda i: (i, 0)),
      out_shape=jax.ShapeDtypeStruct((NP, OUT), jnp.float32),
  )(agg, dd, b2, p1w, p1b, p2w, p2b)


_LR = 256


_PADC = float(NP - N)


def _loss_body(zn1b_ref, zn2b_ref, zn1_ref, zn2_ref, stats_ref, cs_ref):
  i = pl.program_id(0)
  a1 = zn1b_ref[...]
  a2 = zn2b_ref[...]
  z1 = zn1_ref[...]
  z2 = zn2_ref[...]
  dn = (((1,), (1,)), ((), ()))
  inv_t = 1.0 / TEMP
  s11 = lax.dot_general(a1, z1, dn, preferred_element_type=jnp.float32)
  s12 = lax.dot_general(a1, z2, dn, preferred_element_type=jnp.float32)
  s22 = lax.dot_general(a2, z2, dn, preferred_element_type=jnp.float32)
  e11 = jnp.exp(s11 * inv_t)
  e12 = jnp.exp(s12 * inv_t)
  e22 = jnp.exp(s22 * inv_t)
  ones_c = jnp.full((NP, 1), 1.0, jnp.float32)
  rs11 = jnp.dot(e11, ones_c, preferred_element_type=jnp.float32)[:, 0] - _PADC
  rs12 = jnp.dot(e12, ones_c, preferred_element_type=jnp.float32)[:, 0] - _PADC
  rs22 = jnp.dot(e22, ones_c, preferred_element_type=jnp.float32)[:, 0] - _PADC
  d11 = jnp.exp(jnp.sum(a1 * a1, axis=1) * inv_t)
  d12 = jnp.exp(jnp.sum(a1 * a2, axis=1) * inv_t)
  d22 = jnp.exp(jnp.sum(a2 * a2, axis=1) * inv_t)
  stats_ref[...] = jnp.stack(
      [rs11, rs12, rs22, d11, d12, d22, rs11, rs11], axis=0)
  ones_r = jnp.full((1, _LR), 1.0, jnp.float32)
  csum = jnp.dot(ones_r, e12, preferred_element_type=jnp.float32)

  @pl.when(i == 0)
  def _():
    cs_ref[...] = csum

  @pl.when(i > 0)
  def _():
    cs_ref[...] += csum


def _loss_blocks(zn1, zn2):
  return pl.pallas_call(
      _loss_body,
      grid=(NP // _LR,),
      in_specs=[pl.BlockSpec((_LR, OUT), lambda i: (i, 0)),
                pl.BlockSpec((_LR, OUT), lambda i: (i, 0)),
                pl.BlockSpec((NP, OUT), lambda i: (0, 0)),
                pl.BlockSpec((NP, OUT), lambda i: (0, 0))],
      out_specs=[pl.BlockSpec((8, _LR), lambda i: (0, i)),
                 pl.BlockSpec((1, NP), lambda i: (0, 0))],
      out_shape=[jax.ShapeDtypeStruct((8, NP), jnp.float32),
                 jax.ShapeDtypeStruct((1, NP), jnp.float32)],
  )(zn1, zn2, zn1, zn2)


def _reduce_body(stats_ref, cs_ref, o_ref):
  rs11 = stats_ref[0:1, :]
  rs12 = stats_ref[1:2, :]
  rs22 = stats_ref[2:3, :]
  d11 = stats_ref[3:4, :]
  d12 = stats_ref[4:5, :]
  d22 = stats_ref[5:6, :]
  cs12 = cs_ref[...] - _PADC
  x1 = rs11 + rs12 - d11
  x2 = rs22 + cs12 - d22
  l1 = jnp.log(x1) - jnp.log(d12)
  l2 = jnp.log(x2) - jnp.log(d12)
  col = lax.broadcasted_iota(jnp.int32, (1, NP), 1)
  lr = jnp.where(col < N, 0.5 * (l1 + l2), 0.0)
  o_ref[...] = jnp.sum(lr).reshape(1, 1) / N


def _reduce(stats, cs):
  return pl.pallas_call(
      _reduce_body,
      grid=(1,),
      in_specs=[pl.BlockSpec((8, NP), lambda i: (0, 0)),
                pl.BlockSpec((1, NP), lambda i: (0, 0))],
      out_specs=pl.BlockSpec((1, 1), lambda i: (0, 0)),
      out_shape=jax.ShapeDtypeStruct((1, 1), jnp.float32),
  )(stats, cs)


def _pad_edges(ei):
  pad = jnp.full((EP - E,), N, jnp.int32)
  src = jnp.concatenate([ei[0].astype(jnp.int32), pad]).reshape(NTILES, NCH, CH)
  dst = jnp.concatenate([ei[1].astype(jnp.int32), pad]).reshape(NTILES, NCH, CH)
  return jnp.stack([src, dst], axis=2)


def kernel(feat1, feat2, W1, b1, W2, b2, P1w, P1b, P2w, P2b,
           edge_index1, edge_index2):
  f32 = jnp.float32
  e1 = _pad_edges(edge_index1)
  e2 = _pad_edges(edge_index2)
  pad = jnp.zeros((NP - N, D), f32)
  f1 = jnp.concatenate([feat1.astype(f32), pad])
  f2 = jnp.concatenate([feat2.astype(f32), pad])

  zsmall = jnp.zeros((NP, 16), f32)
  ones_h = jnp.ones((CH, 16), f32)
  zbig64 = jnp.zeros((TPW, FH), f32)

  ds1, dd1, ds2, dd2 = _make_deg_kernel()(e1, e2, zsmall, ones_h)

  b1r = b1.reshape(1, H)
  b2r = b2.reshape(1, OUT)
  p1br = P1b.reshape(1, PDIM)
  p2br = P2b.reshape(1, OUT)

  hs1a, hs1b = _scale(f1, ds1)
  hs2a, hs2b = _scale(f2, ds2)
  agg = _make_agg_kernel()
  agg1a, agg2a = agg(hs1a, e1, hs2a, e2, zbig64)
  agg1b, agg2b = agg(hs1b, e1, hs2b, e2, zbig64)
  m1 = _layer1(agg1a, agg1b, dd1, ds1, W1, b1r, W2)
  m2 = _layer1(agg2a, agg2b, dd2, ds2, W1, b1r, W2)

  agg1c, agg2c = agg(m1, e1, m2, e2, zbig64)
  zn1 = _final(agg1c, dd1, b2r, P1w, p1br, P2w, p2br)
  zn2 = _final(agg2c, dd2, b2r, P1w, p1br, P2w, p2br)

  stats, cs = _loss_blocks(zn1, zn2)
  out = _reduce(stats, cs)
  return out[0, 0]


# speedup vs baseline: 1.0288x; 1.0288x over previous
import functools

import jax
import jax.numpy as jnp
from jax import lax
from jax.experimental import pallas as pl
from jax.experimental.pallas import tpu as pltpu
from jax.experimental.pallas import tpu_sc as plsc

N = 10000
E = 320000
D = 128
H = 128
OUT = 64
PDIM = 32
TEMP = 0.5

NP = 10240
NTILES = 16
TPW = NP // NTILES
CH = 128
NCH = 160
EPT = NCH * CH
EP = EPT * NTILES

def _sc_mesh():
  return plsc.VectorSubcoreMesh(
      core_axis_name="c", subcore_axis_name="s", num_cores=2, num_subcores=16)


_DEG_LAG = 4


@functools.cache
def _make_deg_kernel():
  return functools.partial(
      pl.kernel,
      out_type=(jax.ShapeDtypeStruct((NP, 16), jnp.float32),) * 4,
      mesh=_sc_mesh(),
      scratch_types=(
          pltpu.VMEM_SHARED((NP, 16), jnp.float32),
          pltpu.VMEM_SHARED((NP, 16), jnp.float32),
          pltpu.VMEM((NCH, 2, CH), jnp.int32),
          pltpu.VMEM((CH, 16), jnp.float32),
          pltpu.SemaphoreType.DMA,
      ),
  )(_deg_body)


def _deg_body(e1, e2, zsmall, ones_h,
              ds1_o, dd1_o, ds2_o, dd2_o,
              acc_s, acc_d, idxv, onesv, sem):
  c = lax.axis_index("c")
  s = lax.axis_index("s")
  pltpu.sync_copy(zsmall.at[pl.ds(s * TPW, TPW)], acc_s.at[pl.ds(s * TPW, TPW)])
  pltpu.sync_copy(zsmall.at[pl.ds(s * TPW, TPW)], acc_d.at[pl.ds(s * TPW, TPW)])
  pltpu.sync_copy(ones_h, onesv)

  def body(e_h):
    pltpu.sync_copy(e_h.at[s], idxv)
    plsc.subcore_barrier()

    def drain(j):
      pltpu.make_async_copy(onesv, acc_s.at[idxv.at[j, 0]], sem).wait()
      pltpu.make_async_copy(onesv, acc_d.at[idxv.at[j, 1]], sem).wait()

    def step(j, carry):
      pltpu.async_copy(onesv, acc_s.at[idxv.at[j, 0]], sem, add=True)
      pltpu.async_copy(onesv, acc_d.at[idxv.at[j, 1]], sem, add=True)
      pl.when(j >= _DEG_LAG)(lambda: drain(j - _DEG_LAG))
      return carry

    lax.fori_loop(0, NCH, step, 0)

    def tail(j, carry):
      drain(j)
      return carry

    lax.fori_loop(NCH - _DEG_LAG, NCH, tail, 0)

  pl.when(c == 0)(lambda: body(e1))
  pl.when(c == 1)(lambda: body(e2))
  plsc.subcore_barrier()

  def copyout(o_s, o_d):
    pltpu.sync_copy(acc_s.at[pl.ds(s * TPW, TPW)], o_s.at[pl.ds(s * TPW, TPW)])
    pltpu.sync_copy(acc_d.at[pl.ds(s * TPW, TPW)], o_d.at[pl.ds(s * TPW, TPW)])

  pl.when(c == 0)(lambda: copyout(ds1_o, dd1_o))
  pl.when(c == 1)(lambda: copyout(ds2_o, dd2_o))


_NIDX = 8
_NROW = 4
FH = 64


@functools.cache
def _make_agg_kernel():
  return functools.partial(
      pl.kernel,
      out_type=(jax.ShapeDtypeStruct((NP, FH), jnp.float32),) * 2,
      mesh=_sc_mesh(),
      scratch_types=(
          (pltpu.VMEM_SHARED((NP, FH), jnp.float32),
           pltpu.VMEM_SHARED((NP, FH), jnp.float32))
          + tuple(pltpu.VMEM((2, CH), jnp.int32) for _ in range(_NIDX))
          + tuple(pltpu.VMEM((CH, FH), jnp.float32) for _ in range(_NROW))
          + tuple(pltpu.SemaphoreType.DMA for _ in range(_NIDX + _NROW))
      ),
      compiler_params=pltpu.CompilerParams(use_tc_tiling_on_sc=False),
  )(_agg_body)


def _agg_body(hs1, e1, hs2, e2, zbig, out1, out2, tab, acc, *scratch):
  idxb = scratch[:_NIDX]
  bufs = scratch[_NIDX:_NIDX + _NROW]
  sem_i = scratch[_NIDX + _NROW:2 * _NIDX + _NROW]
  sem_g = scratch[2 * _NIDX + _NROW:]
  c = lax.axis_index("c")
  s = lax.axis_index("s")
  rows = pl.ds(s * TPW, TPW)
  pltpu.sync_copy(zbig, acc.at[rows])

  def body(hs, e_h):
    pltpu.sync_copy(hs.at[rows], tab.at[rows])
    plsc.subcore_barrier()
    for b in range(_NIDX):
      pltpu.async_copy(e_h.at[s, b], idxb[b], sem_i[b])
    for j in range(_NROW):
      pltpu.make_async_copy(e_h.at[s, j], idxb[j], sem_i[j]).wait()
      pltpu.async_copy(tab.at[idxb[j].at[0]], bufs[j], sem_g[j])

    def step(q, carry):
      for b in range(_NIDX):
        j = q * _NIDX + b
        rb = b % _NROW
        pltpu.make_async_copy(tab.at[idxb[b].at[0]], bufs[rb], sem_g[rb]).wait()
        pltpu.sync_copy(bufs[rb], acc.at[idxb[b].at[1]], add=True)

        def fire_idx(b=b, j=j):
          pltpu.async_copy(e_h.at[s, j + _NIDX], idxb[b], sem_i[b])

        pl.when(j + _NIDX < NCH)(fire_idx)

        def fire_gather(b=b, j=j, rb=rb):
          b2 = (b + _NROW) % _NIDX
          pltpu.make_async_copy(e_h.at[s, j + _NROW], idxb[b2],
                                sem_i[b2]).wait()
          pltpu.async_copy(tab.at[idxb[b2].at[0]], bufs[rb], sem_g[rb])

        pl.when(j + _NROW < NCH)(fire_gather)
      return carry

    lax.fori_loop(0, NCH // _NIDX, step, 0)

  pl.when(c == 0)(lambda: body(hs1, e1))
  pl.when(c == 1)(lambda: body(hs2, e2))
  plsc.subcore_barrier()

  pl.when(c == 0)(lambda: pltpu.sync_copy(acc.at[rows], out1.at[rows]))
  pl.when(c == 1)(lambda: pltpu.sync_copy(acc.at[rows], out2.at[rows]))


def _rsqdeg(deg_blk):
  return lax.rsqrt(jnp.maximum(deg_blk[:, 0:1], 1.0))


def _scale_body(x_ref, deg_ref, oa_ref, ob_ref):
  h = x_ref[...] * _rsqdeg(deg_ref[...])
  oa_ref[...] = h[:, :FH]
  ob_ref[...] = h[:, FH:]


def _scale(x, deg):
  R = 2048
  return pl.pallas_call(
      _scale_body,
      grid=(NP // R,),
      in_specs=[pl.BlockSpec((R, D), lambda i: (i, 0)),
                pl.BlockSpec((R, 16), lambda i: (i, 0))],
      out_specs=[pl.BlockSpec((R, FH), lambda i: (i, 0)),
                 pl.BlockSpec((R, FH), lambda i: (i, 0))],
      out_shape=[jax.ShapeDtypeStruct((NP, FH), jnp.float32),
                 jax.ShapeDtypeStruct((NP, FH), jnp.float32)],
  )(x, deg)


def _layer1_body(aa_ref, ab_ref, dd_ref, ds_ref, w_ref, b_ref, w2_ref, o_ref):
  nd = _rsqdeg(dd_ref[...])
  h = (jnp.dot(aa_ref[...] * nd, w_ref[:FH, :],
               preferred_element_type=jnp.float32)
       + jnp.dot(ab_ref[...] * nd, w_ref[FH:, :],
                 preferred_element_type=jnp.float32)
       + b_ref[...])
  h = jnp.maximum(h, 0.0) * _rsqdeg(ds_ref[...])
  o_ref[...] = jnp.dot(h, w2_ref[...], preferred_element_type=jnp.float32)


def _layer1(agga, aggb, dd, ds, w, b, w2):
  R = 1024
  return pl.pallas_call(
      _layer1_body,
      grid=(NP // R,),
      in_specs=[pl.BlockSpec((R, FH), lambda i: (i, 0)),
                pl.BlockSpec((R, FH), lambda i: (i, 0)),
                pl.BlockSpec((R, 16), lambda i: (i, 0)),
                pl.BlockSpec((R, 16), lambda i: (i, 0)),
                pl.BlockSpec((D, H), lambda i: (0, 0)),
                pl.BlockSpec((1, H), lambda i: (0, 0)),
                pl.BlockSpec((H, OUT), lambda i: (0, 0))],
      out_specs=pl.BlockSpec((R, OUT), lambda i: (i, 0)),
      out_shape=jax.ShapeDtypeStruct((NP, OUT), jnp.float32),
  )(agga, aggb, dd, ds, w, b, w2)


_FR = 1024


def _final_body(agg_ref, dd_ref, b2_ref, p1w_ref, p1b_ref,
                p2w_ref, p2b_ref, o_ref):
  h = agg_ref[...] * _rsqdeg(dd_ref[...]) + b2_ref[...]
  h = jnp.maximum(h, 0.0)
  u = jnp.dot(h, p1w_ref[...], preferred_element_type=jnp.float32) + p1b_ref[...]
  u = jnp.where(u > 0.0, u, jnp.exp(u) - 1.0)
  z = jnp.dot(u, p2w_ref[...], preferred_element_type=jnp.float32) + p2b_ref[...]
  nrm = jnp.maximum(jnp.sqrt(jnp.sum(z * z, axis=1, keepdims=True)), 1e-12)
  row = lax.broadcasted_iota(jnp.int32, (_FR, OUT), 0) + pl.program_id(0) * _FR
  o_ref[...] = jnp.where(row < N, z / nrm, 0.0)


def _final(agg, dd, b2, p1w, p1b, p2w, p2b):
  R = _FR
  return pl.pallas_call(
      _final_body,
      grid=(NP // R,),
      in_specs=[pl.BlockSpec((R, OUT), lambda i: (i, 0)),
                pl.BlockSpec((R, 16), lambda i: (i, 0)),
                pl.BlockSpec((1, OUT), lambda i: (0, 0)),
                pl.BlockSpec((OUT, PDIM), lambda i: (0, 0)),
                pl.BlockSpec((1, PDIM), lambda i: (0, 0)),
                pl.BlockSpec((PDIM, OUT), lambda i: (0, 0)),
                pl.BlockSpec((1, OUT), lambda i: (0, 0))],
      out_specs=pl.BlockSpec((R, OUT), lambda i: (i, 0)),
      out_shape=jax.ShapeDtypeStruct((NP, OUT), jnp.float32),
  )(agg, dd, b2, p1w, p1b, p2w, p2b)


_LR = 256


_PADC = float(NP - N)


def _loss_body(zn1b_ref, zn2b_ref, zn1_ref, zn2_ref, stats_ref, cs_ref):
  i = pl.program_id(0)
  a1 = zn1b_ref[...]
  a2 = zn2b_ref[...]
  z1 = zn1_ref[...]
  z2 = zn2_ref[...]
  dn = (((1,), (1,)), ((), ()))
  inv_t = 1.0 / TEMP
  s11 = lax.dot_general(a1, z1, dn, preferred_element_type=jnp.float32)
  s12 = lax.dot_general(a1, z2, dn, preferred_element_type=jnp.float32)
  s22 = lax.dot_general(a2, z2, dn, preferred_element_type=jnp.float32)
  e11 = jnp.exp(s11 * inv_t)
  e12 = jnp.exp(s12 * inv_t)
  e22 = jnp.exp(s22 * inv_t)
  ones_c = jnp.full((NP, 1), 1.0, jnp.float32)
  rs11 = jnp.dot(e11, ones_c, preferred_element_type=jnp.float32)[:, 0] - _PADC
  rs12 = jnp.dot(e12, ones_c, preferred_element_type=jnp.float32)[:, 0] - _PADC
  rs22 = jnp.dot(e22, ones_c, preferred_element_type=jnp.float32)[:, 0] - _PADC
  d11 = jnp.exp(jnp.sum(a1 * a1, axis=1) * inv_t)
  d12 = jnp.exp(jnp.sum(a1 * a2, axis=1) * inv_t)
  d22 = jnp.exp(jnp.sum(a2 * a2, axis=1) * inv_t)
  stats_ref[...] = jnp.stack(
      [rs11, rs12, rs22, d11, d12, d22, rs11, rs11], axis=0)
  csum = jnp.sum(e12, axis=0, keepdims=True)

  @pl.when(i == 0)
  def _():
    cs_ref[...] = csum

  @pl.when(i > 0)
  def _():
    cs_ref[...] += csum


def _loss_blocks(zn1, zn2):
  return pl.pallas_call(
      _loss_body,
      grid=(NP // _LR,),
      in_specs=[pl.BlockSpec((_LR, OUT), lambda i: (i, 0)),
                pl.BlockSpec((_LR, OUT), lambda i: (i, 0)),
                pl.BlockSpec((NP, OUT), lambda i: (0, 0)),
                pl.BlockSpec((NP, OUT), lambda i: (0, 0))],
      out_specs=[pl.BlockSpec((8, _LR), lambda i: (0, i)),
                 pl.BlockSpec((1, NP), lambda i: (0, 0))],
      out_shape=[jax.ShapeDtypeStruct((8, NP), jnp.float32),
                 jax.ShapeDtypeStruct((1, NP), jnp.float32)],
  )(zn1, zn2, zn1, zn2)


def _reduce_body(stats_ref, cs_ref, o_ref):
  rs11 = stats_ref[0:1, :]
  rs12 = stats_ref[1:2, :]
  rs22 = stats_ref[2:3, :]
  d11 = stats_ref[3:4, :]
  d12 = stats_ref[4:5, :]
  d22 = stats_ref[5:6, :]
  cs12 = cs_ref[...] - _PADC
  x1 = rs11 + rs12 - d11
  x2 = rs22 + cs12 - d22
  l1 = jnp.log(x1) - jnp.log(d12)
  l2 = jnp.log(x2) - jnp.log(d12)
  col = lax.broadcasted_iota(jnp.int32, (1, NP), 1)
  lr = jnp.where(col < N, 0.5 * (l1 + l2), 0.0)
  o_ref[...] = jnp.sum(lr).reshape(1, 1) / N


def _reduce(stats, cs):
  return pl.pallas_call(
      _reduce_body,
      grid=(1,),
      in_specs=[pl.BlockSpec((8, NP), lambda i: (0, 0)),
                pl.BlockSpec((1, NP), lambda i: (0, 0))],
      out_specs=pl.BlockSpec((1, 1), lambda i: (0, 0)),
      out_shape=jax.ShapeDtypeStruct((1, 1), jnp.float32),
  )(stats, cs)


def _pad_edges(ei):
  pad = jnp.full((EP - E,), N, jnp.int32)
  src = jnp.concatenate([ei[0].astype(jnp.int32), pad]).reshape(NTILES, NCH, CH)
  dst = jnp.concatenate([ei[1].astype(jnp.int32), pad]).reshape(NTILES, NCH, CH)
  return jnp.stack([src, dst], axis=2)


def kernel(feat1, feat2, W1, b1, W2, b2, P1w, P1b, P2w, P2b,
           edge_index1, edge_index2):
  f32 = jnp.float32
  e1 = _pad_edges(edge_index1)
  e2 = _pad_edges(edge_index2)
  pad = jnp.zeros((NP - N, D), f32)
  f1 = jnp.concatenate([feat1.astype(f32), pad])
  f2 = jnp.concatenate([feat2.astype(f32), pad])

  zsmall = jnp.zeros((NP, 16), f32)
  ones_h = jnp.ones((CH, 16), f32)
  zbig64 = jnp.zeros((TPW, FH), f32)

  ds1, dd1, ds2, dd2 = _make_deg_kernel()(e1, e2, zsmall, ones_h)

  b1r = b1.reshape(1, H)
  b2r = b2.reshape(1, OUT)
  p1br = P1b.reshape(1, PDIM)
  p2br = P2b.reshape(1, OUT)

  hs1a, hs1b = _scale(f1, ds1)
  hs2a, hs2b = _scale(f2, ds2)
  agg = _make_agg_kernel()
  agg1a, agg2a = agg(hs1a, e1, hs2a, e2, zbig64)
  agg1b, agg2b = agg(hs1b, e1, hs2b, e2, zbig64)
  m1 = _layer1(agg1a, agg1b, dd1, ds1, W1, b1r, W2)
  m2 = _layer1(agg2a, agg2b, dd2, ds2, W1, b1r, W2)

  agg1c, agg2c = agg(m1, e1, m2, e2, zbig64)
  zn1 = _final(agg1c, dd1, b2r, P1w, p1br, P2w, p2br)
  zn2 = _final(agg2c, dd2, b2r, P1w, p1br, P2w, p2br)

  stats, cs = _loss_blocks(zn1, zn2)
  out = _reduce(stats, cs)
  return out[0, 0]
